# Initial kernel scaffold; baseline (speedup 1.0000x reference)
import jax
import jax.numpy as jnp
from jax.experimental import pallas as pl


def kernel(x, edge_index, edge_attr, W1, b1, W2, b2):
    raise NotImplementedError("write your pallas kernel here")


# speedup vs baseline: 22.7958x; 22.7958x over previous
import functools

import jax
import jax.numpy as jnp
from jax import lax
from jax.experimental import pallas as pl
from jax.experimental.pallas import tpu as pltpu
from jax.experimental.pallas import tpu_sc as plsc

N_NODES = 10000
D = 16
SUB = 128
CHUNK = 2048
NSUB = CHUNK // SUB
NW = 32
EPT = 10240
E_PAD = NW * EPT
NCHUNK = EPT // CHUNK
N_PAD = 10240
RPT = N_PAD // 16


def _make_agg():
    mesh = plsc.VectorSubcoreMesh(core_axis_name="c", subcore_axis_name="s")

    @functools.partial(
        pl.kernel,
        mesh=mesh,
        compiler_params=pltpu.CompilerParams(use_tc_tiling_on_sc=False),
        out_type=jax.ShapeDtypeStruct((2, N_PAD, D), jnp.float32),
        scratch_types=[
            pltpu.VMEM((NSUB, SUB), jnp.int32),
            pltpu.VMEM((NSUB, SUB), jnp.int32),
            pltpu.VMEM((CHUNK,), jnp.float32),
            pltpu.VMEM((CHUNK, D), jnp.float32),
            pltpu.VMEM_SHARED((N_PAD, D), jnp.float32),
            pltpu.SemaphoreType.DMA,
        ],
    )
    def agg(g_hbm, src_hbm, dst_hbm, ew_hbm, zeros_hbm, out_hbm,
            srcv, dstv, eww, rows, acc_sh, sem):
        c = lax.axis_index("c")
        s = lax.axis_index("s")
        w = c * 16 + s
        pltpu.sync_copy(zeros_hbm.at[pl.ds(s * RPT, RPT)],
                        acc_sh.at[pl.ds(s * RPT, RPT)])
        plsc.subcore_barrier()
        for ci in range(NCHUNK):
            row0 = w * (EPT // SUB) + ci * NSUB
            lin0 = w * EPT + ci * CHUNK
            pltpu.sync_copy(src_hbm.at[pl.ds(row0, NSUB)], srcv)
            pltpu.sync_copy(dst_hbm.at[pl.ds(row0, NSUB)], dstv)
            pltpu.sync_copy(ew_hbm.at[pl.ds(lin0, CHUNK)], eww)
            cps = [pltpu.async_copy(g_hbm.at[srcv.at[j]],
                                    rows.at[pl.ds(j * SUB, SUB)], sem)
                   for j in range(NSUB)]
            for cp in cps:
                cp.wait()

            def body(gi, _):
                base = gi * 16
                ewv = eww[pl.ds(base, 16)]
                for j in range(16):
                    wv = jnp.broadcast_to(lax.slice(ewv, (j,), (j + 1,)), (16,))
                    rows[base + j, :] = rows[base + j, :] * wv
                return 0

            lax.fori_loop(0, CHUNK // 16, body, 0)
            for j in range(NSUB):
                pltpu.sync_copy(rows.at[pl.ds(j * SUB, SUB)],
                                acc_sh.at[dstv.at[j]], add=True)
        plsc.subcore_barrier()
        pltpu.sync_copy(acc_sh.at[pl.ds(s * RPT, RPT)],
                        out_hbm.at[c].at[pl.ds(s * RPT, RPT)])

    return agg


_AGG = _make_agg()

_BR = 1000


def _tc_layer1(x, W1, degp):
    def body(x_ref, w_ref, degp_ref, h1_ref, g1_ref, dinv_ref):
        dinvb = lax.rsqrt(degp_ref[0] + degp_ref[1] + 1.0)
        h1 = jnp.dot(x_ref[...], w_ref[...], preferred_element_type=jnp.float32)
        h1_ref[...] = h1
        g1_ref[...] = h1 * dinvb
        dinv_ref[...] = dinvb

    return pl.pallas_call(
        body,
        grid=(N_NODES // _BR,),
        in_specs=[
            pl.BlockSpec((_BR, 128), lambda i: (i, 0)),
            pl.BlockSpec((128, D), lambda i: (0, 0)),
            pl.BlockSpec((2, _BR, D), lambda i: (0, i, 0)),
        ],
        out_specs=[pl.BlockSpec((_BR, D), lambda i: (i, 0))] * 3,
        out_shape=[jax.ShapeDtypeStruct((N_NODES, D), jnp.float32)] * 3,
    )(x, W1, degp)


def _tc_layer2(accp, h1, dinvb, b1, W2pad):
    def body(acc_ref, h1_ref, dinv_ref, b1_ref, w2_ref, h2_ref, g2_ref):
        dv = dinv_ref[...]
        pre = dv * (acc_ref[0] + acc_ref[1]) + dv * dv * h1_ref[...] + b1_ref[...]
        h = jnp.maximum(pre, 0.0)
        h2 = jnp.dot(h, w2_ref[...], preferred_element_type=jnp.float32)
        h2_ref[...] = h2
        g2_ref[...] = h2 * dv

    return pl.pallas_call(
        body,
        grid=(N_NODES // _BR,),
        in_specs=[
            pl.BlockSpec((2, _BR, D), lambda i: (0, i, 0)),
            pl.BlockSpec((_BR, D), lambda i: (i, 0)),
            pl.BlockSpec((_BR, D), lambda i: (i, 0)),
            pl.BlockSpec((1, D), lambda i: (0, 0)),
            pl.BlockSpec((D, D), lambda i: (0, 0)),
        ],
        out_specs=[pl.BlockSpec((_BR, D), lambda i: (i, 0))] * 2,
        out_shape=[jax.ShapeDtypeStruct((N_NODES, D), jnp.float32)] * 2,
    )(accp, h1, dinvb, b1, W2pad)


def _tc_final(accp, h2, dinvb, b2pad):
    def body(acc_ref, h2_ref, dinv_ref, b2_ref, out_ref):
        dv = dinv_ref[...]
        out_ref[...] = (dv * (acc_ref[0] + acc_ref[1])
                        + dv * dv * h2_ref[...] + b2_ref[...])

    return pl.pallas_call(
        body,
        grid=(N_NODES // _BR,),
        in_specs=[
            pl.BlockSpec((2, _BR, D), lambda i: (0, i, 0)),
            pl.BlockSpec((_BR, D), lambda i: (i, 0)),
            pl.BlockSpec((_BR, D), lambda i: (i, 0)),
            pl.BlockSpec((1, D), lambda i: (0, 0)),
        ],
        out_specs=pl.BlockSpec((_BR, D), lambda i: (i, 0)),
        out_shape=jax.ShapeDtypeStruct((N_NODES, D), jnp.float32),
    )(accp, h2, dinvb, b2pad)


def kernel(x, edge_index, edge_attr, W1, b1, W2, b2):
    src = edge_index[0]
    dst = edge_index[1]
    ew = edge_attr.reshape(-1)
    npad = E_PAD - src.shape[0]
    srcp = jnp.concatenate([src, jnp.zeros((npad,), src.dtype)])
    dstp = jnp.concatenate([dst, jnp.zeros((npad,), dst.dtype)])
    ewp = jnp.concatenate([ew, jnp.zeros((npad,), ew.dtype)])
    src2d = srcp.reshape(E_PAD // SUB, SUB)
    dst2d = dstp.reshape(E_PAD // SUB, SUB)
    ones = jnp.ones((N_NODES, D), jnp.float32)
    zeros = jnp.zeros((N_PAD, D), jnp.float32)

    degp = _AGG(ones, src2d, dst2d, ewp, zeros)[:, :N_NODES]
    h1, g1, dinvb = _tc_layer1(x, W1, degp)
    acc1 = _AGG(g1, src2d, dst2d, ewp, zeros)[:, :N_NODES]
    W2pad = jnp.pad(W2, ((0, 0), (0, D - W2.shape[1])))
    h2, g2 = _tc_layer2(acc1, h1, dinvb, b1.reshape(1, D), W2pad)
    acc2 = _AGG(g2, src2d, dst2d, ewp, zeros)[:, :N_NODES]
    b2pad = jnp.pad(b2, (0, D - b2.shape[0])).reshape(1, D)
    out16 = _tc_final(acc2, h2, dinvb, b2pad)
    return out16[:, :b2.shape[0]]


# speedup vs baseline: 31.5931x; 1.3859x over previous
import functools

import jax
import jax.numpy as jnp
from jax import lax
from jax.experimental import pallas as pl
from jax.experimental.pallas import tpu as pltpu
from jax.experimental.pallas import tpu_sc as plsc

N_NODES = 10000
D = 16
SUB = 128
CHUNK = 2048
NSUB = CHUNK // SUB
NW = 32
EPT = 10240
E_PAD = NW * EPT
NCHUNK = EPT // CHUNK
N_PAD = 10240
RPT = N_PAD // 16


def _make_agg(with_gather):
    mesh = plsc.VectorSubcoreMesh(core_axis_name="c", subcore_axis_name="s")

    def agg(*args):
        if with_gather:
            (g_hbm, src_hbm, dst_hbm, ew_hbm, zeros_hbm, out_hbm,
             srcv, dstv, eww, rows, acc_sh, sg0, sg1, ss0, ss1) = args
        else:
            (dst_hbm, ew_hbm, zeros_hbm, out_hbm,
             srcv, dstv, eww, rows, acc_sh, sg0, sg1, ss0, ss1) = args
        c = lax.axis_index("c")
        s = lax.axis_index("s")
        w = c * 16 + s
        pltpu.sync_copy(zeros_hbm.at[pl.ds(s * RPT, RPT)],
                        acc_sh.at[pl.ds(s * RPT, RPT)])
        plsc.subcore_barrier()
        sg = [sg0, sg1]
        ss = [ss0, ss1]
        gh = [[], []]
        sh = [[], []]

        def load_idx(ci, b):
            row0 = w * (EPT // SUB) + ci * NSUB
            lin0 = w * EPT + ci * CHUNK
            if with_gather:
                pltpu.sync_copy(src_hbm.at[pl.ds(row0, NSUB)], srcv.at[b])
            pltpu.sync_copy(dst_hbm.at[pl.ds(row0, NSUB)], dstv.at[b])
            pltpu.sync_copy(ew_hbm.at[pl.ds(lin0, CHUNK)], eww.at[b])

        def fire_gathers(b):
            if with_gather:
                gh[b] = [pltpu.async_copy(g_hbm.at[srcv.at[b, j]],
                                          rows.at[b, pl.ds(j * SUB, SUB)],
                                          sg[b])
                         for j in range(NSUB)]

        def fire_scatters(b):
            sh[b] = [pltpu.async_copy(rows.at[b, pl.ds(j * SUB, SUB)],
                                      acc_sh.at[dstv.at[b, j]], ss[b],
                                      add=True)
                     for j in range(NSUB)]

        load_idx(0, 0)
        fire_gathers(0)
        for ci in range(NCHUNK):
            b = ci % 2
            nb = 1 - b
            if ci + 1 < NCHUNK:
                for hnd in sh[nb]:
                    hnd.wait()
                sh[nb] = []
                load_idx(ci + 1, nb)
                fire_gathers(nb)
            for hnd in gh[b]:
                hnd.wait()
            gh[b] = []

            def body(gi, _):
                base = gi * 16
                ewv = eww[b, pl.ds(base, 16)]
                for j in range(16):
                    wv = jnp.broadcast_to(lax.slice(ewv, (j,), (j + 1,)), (16,))
                    if with_gather:
                        rows[b, base + j, :] = rows[b, base + j, :] * wv
                    else:
                        rows[b, base + j, :] = wv
                return 0

            lax.fori_loop(0, CHUNK // 16, body, 0)
            fire_scatters(b)
        for b in (0, 1):
            for hnd in sh[b]:
                hnd.wait()
        plsc.subcore_barrier()
        pltpu.sync_copy(acc_sh.at[pl.ds(s * RPT, RPT)],
                        out_hbm.at[c].at[pl.ds(s * RPT, RPT)])

    return pl.kernel(
        agg,
        mesh=mesh,
        compiler_params=pltpu.CompilerParams(use_tc_tiling_on_sc=False),
        out_type=jax.ShapeDtypeStruct((2, N_PAD, D), jnp.float32),
        scratch_types=[
            pltpu.VMEM((2, NSUB, SUB), jnp.int32),
            pltpu.VMEM((2, NSUB, SUB), jnp.int32),
            pltpu.VMEM((2, CHUNK), jnp.float32),
            pltpu.VMEM((2, CHUNK, D), jnp.float32),
            pltpu.VMEM_SHARED((N_PAD, D), jnp.float32),
            pltpu.SemaphoreType.DMA,
            pltpu.SemaphoreType.DMA,
            pltpu.SemaphoreType.DMA,
            pltpu.SemaphoreType.DMA,
        ],
    )


_AGG = _make_agg(True)
_DEG = _make_agg(False)

_BR = 1000


def _tc_layer1(x, W1, degp):
    def body(x_ref, w_ref, degp_ref, h1_ref, g1_ref, dinv_ref):
        dinvb = lax.rsqrt(degp_ref[0] + degp_ref[1] + 1.0)
        h1 = jnp.dot(x_ref[...], w_ref[...], preferred_element_type=jnp.float32)
        h1_ref[...] = h1
        g1_ref[...] = h1 * dinvb
        dinv_ref[...] = dinvb

    return pl.pallas_call(
        body,
        grid=(N_NODES // _BR,),
        in_specs=[
            pl.BlockSpec((_BR, 128), lambda i: (i, 0)),
            pl.BlockSpec((128, D), lambda i: (0, 0)),
            pl.BlockSpec((2, _BR, D), lambda i: (0, i, 0)),
        ],
        out_specs=[pl.BlockSpec((_BR, D), lambda i: (i, 0))] * 3,
        out_shape=[jax.ShapeDtypeStruct((N_NODES, D), jnp.float32)] * 3,
    )(x, W1, degp)


def _tc_layer2(accp, h1, dinvb, b1, W2pad):
    def body(acc_ref, h1_ref, dinv_ref, b1_ref, w2_ref, h2_ref, g2_ref):
        dv = dinv_ref[...]
        pre = dv * (acc_ref[0] + acc_ref[1]) + dv * dv * h1_ref[...] + b1_ref[...]
        h = jnp.maximum(pre, 0.0)
        h2 = jnp.dot(h, w2_ref[...], preferred_element_type=jnp.float32)
        h2_ref[...] = h2
        g2_ref[...] = h2 * dv

    return pl.pallas_call(
        body,
        grid=(N_NODES // _BR,),
        in_specs=[
            pl.BlockSpec((2, _BR, D), lambda i: (0, i, 0)),
            pl.BlockSpec((_BR, D), lambda i: (i, 0)),
            pl.BlockSpec((_BR, D), lambda i: (i, 0)),
            pl.BlockSpec((1, D), lambda i: (0, 0)),
            pl.BlockSpec((D, D), lambda i: (0, 0)),
        ],
        out_specs=[pl.BlockSpec((_BR, D), lambda i: (i, 0))] * 2,
        out_shape=[jax.ShapeDtypeStruct((N_NODES, D), jnp.float32)] * 2,
    )(accp, h1, dinvb, b1, W2pad)


def _tc_final(accp, h2, dinvb, b2pad):
    def body(acc_ref, h2_ref, dinv_ref, b2_ref, out_ref):
        dv = dinv_ref[...]
        out_ref[...] = (dv * (acc_ref[0] + acc_ref[1])
                        + dv * dv * h2_ref[...] + b2_ref[...])

    return pl.pallas_call(
        body,
        grid=(N_NODES // _BR,),
        in_specs=[
            pl.BlockSpec((2, _BR, D), lambda i: (0, i, 0)),
            pl.BlockSpec((_BR, D), lambda i: (i, 0)),
            pl.BlockSpec((_BR, D), lambda i: (i, 0)),
            pl.BlockSpec((1, D), lambda i: (0, 0)),
        ],
        out_specs=pl.BlockSpec((_BR, D), lambda i: (i, 0)),
        out_shape=jax.ShapeDtypeStruct((N_NODES, D), jnp.float32),
    )(accp, h2, dinvb, b2pad)


def kernel(x, edge_index, edge_attr, W1, b1, W2, b2):
    src = edge_index[0]
    dst = edge_index[1]
    ew = edge_attr.reshape(-1)
    npad = E_PAD - src.shape[0]
    srcp = jnp.concatenate([src, jnp.zeros((npad,), src.dtype)])
    dstp = jnp.concatenate([dst, jnp.zeros((npad,), dst.dtype)])
    ewp = jnp.concatenate([ew, jnp.zeros((npad,), ew.dtype)])
    src2d = srcp.reshape(E_PAD // SUB, SUB)
    dst2d = dstp.reshape(E_PAD // SUB, SUB)
    zeros = jnp.zeros((N_PAD, D), jnp.float32)

    degp = _DEG(dst2d, ewp, zeros)[:, :N_NODES]
    h1, g1, dinvb = _tc_layer1(x, W1, degp)
    acc1 = _AGG(g1, src2d, dst2d, ewp, zeros)[:, :N_NODES]
    W2pad = jnp.pad(W2, ((0, 0), (0, D - W2.shape[1])))
    h2, g2 = _tc_layer2(acc1, h1, dinvb, b1.reshape(1, D), W2pad)
    acc2 = _AGG(g2, src2d, dst2d, ewp, zeros)[:, :N_NODES]
    b2pad = jnp.pad(b2, (0, D - b2.shape[0])).reshape(1, D)
    out16 = _tc_final(acc2, h2, dinvb, b2pad)
    return out16[:, :b2.shape[0]]
